# Initial kernel scaffold; baseline (speedup 1.0000x reference)
import functools

import jax
import jax.numpy as jnp
from jax import lax
from jax.experimental import pallas as pl
from jax.experimental.pallas import tpu as pltpu
from jax.experimental.pallas import tpu_sc as plsc

N_NODES = 10000
DIM = 256
HALF = 128
N_EDGES = 160000
NC = 2
NS = 16

CHUNK = 80
EDGES_PER_TILE = N_EDGES // NS
CHUNKS_PER_TILE = EDGES_PER_TILE // CHUNK
ROWS_PER_TILE = N_NODES // NS

_sc_mesh = plsc.VectorSubcoreMesh(
    core_axis_name="c", subcore_axis_name="s", num_cores=NC, num_subcores=NS
)


@functools.partial(
    pl.kernel,
    out_type=(
        jax.ShapeDtypeStruct((N_NODES, HALF), jnp.float32),
        jax.ShapeDtypeStruct((N_NODES, HALF), jnp.float32),
    ),
    mesh=_sc_mesh,
    scratch_types=[
        pltpu.VMEM_SHARED((N_NODES, HALF), jnp.float32),
        pltpu.VMEM((CHUNK,), jnp.int32),
        pltpu.VMEM((CHUNK,), jnp.int32),
        pltpu.VMEM((CHUNK, HALF), jnp.float32),
        pltpu.SemaphoreType.DMA,
    ],
)
def _sc_aggregate(x_lo, x_hi, row_hbm, col_hbm, zeros_hbm,
                  agg_lo, agg_hi, acc, colv, rowv, rows_v, sem):
    c = lax.axis_index("c")
    s = lax.axis_index("s")
    row0 = s * ROWS_PER_TILE

    pltpu.sync_copy(zeros_hbm.at[pl.ds(row0, ROWS_PER_TILE)],
                    acc.at[pl.ds(row0, ROWS_PER_TILE)])
    plsc.subcore_barrier()

    ebase = s * EDGES_PER_TILE

    def chunk_body(k, x_half):
        base = ebase + k * CHUNK
        pltpu.sync_copy(col_hbm.at[pl.ds(base, CHUNK)], colv)
        pltpu.async_copy(x_half.at[colv], rows_v, sem).wait()
        pltpu.sync_copy(row_hbm.at[pl.ds(base, CHUNK)], rowv)
        pltpu.sync_copy(rows_v, acc.at[rowv], add=True)

    @pl.when(c == 0)
    def _():
        lax.fori_loop(0, CHUNKS_PER_TILE,
                      lambda k, _: (chunk_body(k, x_lo), 0)[1], 0)

    @pl.when(c == 1)
    def _():
        lax.fori_loop(0, CHUNKS_PER_TILE,
                      lambda k, _: (chunk_body(k, x_hi), 0)[1], 0)

    plsc.subcore_barrier()

    @pl.when(c == 0)
    def _():
        pltpu.sync_copy(acc.at[pl.ds(row0, ROWS_PER_TILE)],
                        agg_lo.at[pl.ds(row0, ROWS_PER_TILE)])

    @pl.when(c == 1)
    def _():
        pltpu.sync_copy(acc.at[pl.ds(row0, ROWS_PER_TILE)],
                        agg_hi.at[pl.ds(row0, ROWS_PER_TILE)])


BR = 1000


def _tc_body(x_ref, alo_ref, ahi_ref, ws_ref, wn_ref, bs_ref, bn_ref, o_ref):
    dn = (((1,), (1,)), ((), ()))
    h = lax.dot_general(x_ref[...], ws_ref[...], dn,
                        preferred_element_type=jnp.float32)
    wn = wn_ref[...]
    h = h + lax.dot_general(alo_ref[...], wn[:, :HALF], dn,
                            preferred_element_type=jnp.float32)
    h = h + lax.dot_general(ahi_ref[...], wn[:, HALF:], dn,
                            preferred_element_type=jnp.float32)
    o_ref[...] = jnp.maximum(h + bs_ref[...] + bn_ref[...], 0.0)


_tc_linear = pl.pallas_call(
    _tc_body,
    grid=(N_NODES // BR,),
    in_specs=[
        pl.BlockSpec((BR, DIM), lambda i: (i, 0)),
        pl.BlockSpec((BR, HALF), lambda i: (i, 0)),
        pl.BlockSpec((BR, HALF), lambda i: (i, 0)),
        pl.BlockSpec((DIM, DIM), lambda i: (0, 0)),
        pl.BlockSpec((DIM, DIM), lambda i: (0, 0)),
        pl.BlockSpec((1, DIM), lambda i: (0, 0)),
        pl.BlockSpec((1, DIM), lambda i: (0, 0)),
    ],
    out_specs=pl.BlockSpec((BR, DIM), lambda i: (i, 0)),
    out_shape=jax.ShapeDtypeStruct((N_NODES, DIM), jnp.float32),
)


def kernel(x, edge_index, W_self, b_self, W_neigh, b_neigh):
    ei = edge_index.astype(jnp.int32)
    row = ei[0]
    col = ei[1]
    x_lo = x[:, :HALF]
    x_hi = x[:, HALF:]
    zeros = jnp.zeros((N_NODES, HALF), jnp.float32)
    agg_lo, agg_hi = _sc_aggregate(x_lo, x_hi, row, col, zeros)
    return _tc_linear(x, agg_lo, agg_hi, W_self, W_neigh,
                      b_self.reshape(1, DIM), b_neigh.reshape(1, DIM))


# speedup vs baseline: 3.7163x; 3.7163x over previous
import functools

import jax
import jax.numpy as jnp
from jax import lax
from jax.experimental import pallas as pl
from jax.experimental.pallas import tpu as pltpu
from jax.experimental.pallas import tpu_sc as plsc

N_NODES = 10000
DIM = 256
HALF = 128
N_EDGES = 160000
NC = 2
NS = 16

CHUNK = 80
EDGES_PER_TILE = N_EDGES // NS
CHUNKS_PER_TILE = EDGES_PER_TILE // CHUNK
ROWS_PER_TILE = N_NODES // NS

_sc_mesh = plsc.VectorSubcoreMesh(
    core_axis_name="c", subcore_axis_name="s", num_cores=NC, num_subcores=NS
)


@functools.partial(
    pl.kernel,
    out_type=(
        jax.ShapeDtypeStruct((N_NODES, HALF), jnp.float32),
        jax.ShapeDtypeStruct((N_NODES, HALF), jnp.float32),
    ),
    mesh=_sc_mesh,
    scratch_types=[
        pltpu.VMEM_SHARED((N_NODES, HALF), jnp.float32),
        pltpu.VMEM((CHUNK,), jnp.int32),
        pltpu.VMEM((CHUNK,), jnp.int32),
        pltpu.VMEM((CHUNK, HALF), jnp.float32),
        pltpu.SemaphoreType.DMA,
    ],
)
def _sc_aggregate(x_lo, x_hi, row_hbm, col_hbm, zeros_hbm,
                  agg_lo, agg_hi, acc, colv, rowv, rows_v, sem):
    c = lax.axis_index("c")
    s = lax.axis_index("s")

    @pl.when(s == 0)
    def _():
        pltpu.sync_copy(zeros_hbm, acc)

    plsc.subcore_barrier()

    ebase = s * EDGES_PER_TILE

    def chunk_body(k, x_half):
        base = ebase + k * CHUNK
        pltpu.sync_copy(col_hbm.at[pl.ds(base, CHUNK)], colv)
        pltpu.async_copy(x_half.at[colv], rows_v, sem).wait()
        pltpu.sync_copy(row_hbm.at[pl.ds(base, CHUNK)], rowv)
        pltpu.sync_copy(rows_v, acc.at[rowv], add=True)

    @pl.when(c == 0)
    def _():
        lax.fori_loop(0, CHUNKS_PER_TILE,
                      lambda k, _: (chunk_body(k, x_lo), 0)[1], 0)

    @pl.when(c == 1)
    def _():
        lax.fori_loop(0, CHUNKS_PER_TILE,
                      lambda k, _: (chunk_body(k, x_hi), 0)[1], 0)

    plsc.subcore_barrier()

    @pl.when((s == 0) & (c == 0))
    def _():
        pltpu.sync_copy(acc, agg_lo)

    @pl.when((s == 0) & (c == 1))
    def _():
        pltpu.sync_copy(acc, agg_hi)


BR = 1000


def _tc_body(x_ref, alo_ref, ahi_ref, ws_ref, wn_ref, bs_ref, bn_ref, o_ref):
    dn = (((1,), (1,)), ((), ()))
    h = lax.dot_general(x_ref[...], ws_ref[...], dn,
                        preferred_element_type=jnp.float32)
    wn = wn_ref[...]
    h = h + lax.dot_general(alo_ref[...], wn[:, :HALF], dn,
                            preferred_element_type=jnp.float32)
    h = h + lax.dot_general(ahi_ref[...], wn[:, HALF:], dn,
                            preferred_element_type=jnp.float32)
    o_ref[...] = jnp.maximum(h + bs_ref[...] + bn_ref[...], 0.0)


_tc_linear = pl.pallas_call(
    _tc_body,
    grid=(N_NODES // BR,),
    in_specs=[
        pl.BlockSpec((BR, DIM), lambda i: (i, 0)),
        pl.BlockSpec((BR, HALF), lambda i: (i, 0)),
        pl.BlockSpec((BR, HALF), lambda i: (i, 0)),
        pl.BlockSpec((DIM, DIM), lambda i: (0, 0)),
        pl.BlockSpec((DIM, DIM), lambda i: (0, 0)),
        pl.BlockSpec((1, DIM), lambda i: (0, 0)),
        pl.BlockSpec((1, DIM), lambda i: (0, 0)),
    ],
    out_specs=pl.BlockSpec((BR, DIM), lambda i: (i, 0)),
    out_shape=jax.ShapeDtypeStruct((N_NODES, DIM), jnp.float32),
)


def kernel(x, edge_index, W_self, b_self, W_neigh, b_neigh):
    ei = edge_index.astype(jnp.int32)
    row = ei[0]
    col = ei[1]
    x_lo = x[:, :HALF]
    x_hi = x[:, HALF:]
    zeros = jnp.zeros((N_NODES, HALF), jnp.float32)
    agg_lo, agg_hi = _sc_aggregate(x_lo, x_hi, row, col, zeros)
    return _tc_linear(x, agg_lo, agg_hi, W_self, W_neigh,
                      b_self.reshape(1, DIM), b_neigh.reshape(1, DIM))


# speedup vs baseline: 9.4538x; 2.5439x over previous
import functools

import jax
import jax.numpy as jnp
from jax import lax
from jax.experimental import pallas as pl
from jax.experimental.pallas import tpu as pltpu
from jax.experimental.pallas import tpu_sc as plsc

N_NODES = 10000
DIM = 256
HALF = 128
N_EDGES = 160000
NC = 2
NS = 16

CHUNK = 40
EDGES_PER_TILE = N_EDGES // NS
CHUNKS_PER_TILE = EDGES_PER_TILE // CHUNK
ROWS_PER_TILE = N_NODES // NS

_sc_mesh = plsc.VectorSubcoreMesh(
    core_axis_name="c", subcore_axis_name="s", num_cores=NC, num_subcores=NS
)


NBUF = 5


@functools.partial(
    pl.kernel,
    out_type=(
        jax.ShapeDtypeStruct((N_NODES, HALF), jnp.float32),
        jax.ShapeDtypeStruct((N_NODES, HALF), jnp.float32),
    ),
    mesh=_sc_mesh,
    scratch_types=(
        [
            pltpu.VMEM_SHARED((N_NODES, HALF), jnp.float32),
            pltpu.VMEM((EDGES_PER_TILE,), jnp.int32),
        ]
        + [pltpu.VMEM((CHUNK, HALF), jnp.float32)] * NBUF
        + [pltpu.VMEM((CHUNK,), jnp.int32)] * NBUF
        + [pltpu.SemaphoreType.DMA] * (3 * NBUF)
    ),
)
def _sc_aggregate(x_lo, x_hi, row_hbm, col_hbm, zeros_hbm,
                  agg_lo, agg_hi, acc, col1d, *bufs_and_sems):
    rows = bufs_and_sems[:NBUF]
    rbufs = bufs_and_sems[NBUF:2 * NBUF]
    rsems = bufs_and_sems[2 * NBUF:3 * NBUF]
    gsems = bufs_and_sems[3 * NBUF:4 * NBUF]
    ssems = bufs_and_sems[4 * NBUF:]
    c = lax.axis_index("c")
    s = lax.axis_index("s")
    ebase = s * EDGES_PER_TILE

    pltpu.sync_copy(col_hbm.at[pl.ds(ebase, EDGES_PER_TILE)], col1d)

    @pl.when(s == 0)
    def _():
        pltpu.sync_copy(zeros_hbm, acc)

    plsc.subcore_barrier()

    def run(x_half):
        def issue(k, b):
            pltpu.async_copy(row_hbm.at[pl.ds(ebase + k * CHUNK, CHUNK)],
                             rbufs[b], rsems[b])
            pltpu.async_copy(x_half.at[col1d.at[pl.ds(k * CHUNK, CHUNK)]],
                             rows[b], gsems[b])

        for b in range(NBUF):
            issue(b, b)

        def outer(g, _):
            for b in range(NBUF):
                k = g * NBUF + b
                pltpu.make_async_copy(x_half.at[col1d.at[pl.ds(0, CHUNK)]],
                                      rows[b], gsems[b]).wait()
                pltpu.make_async_copy(row_hbm.at[pl.ds(0, CHUNK)],
                                      rbufs[b], rsems[b]).wait()
                pltpu.async_copy(rows[b], acc.at[rbufs[b]], ssems[b],
                                 add=True)
                pltpu.make_async_copy(rows[b], acc.at[rbufs[b]],
                                      ssems[b]).wait()

                @pl.when(k + NBUF < CHUNKS_PER_TILE)
                def _():
                    issue(k + NBUF, b)
            return 0

        lax.fori_loop(0, CHUNKS_PER_TILE // NBUF, outer, 0)

    @pl.when(c == 0)
    def _():
        run(x_lo)

    @pl.when(c == 1)
    def _():
        run(x_hi)

    plsc.subcore_barrier()

    @pl.when((s == 0) & (c == 0))
    def _():
        pltpu.sync_copy(acc, agg_lo)

    @pl.when((s == 0) & (c == 1))
    def _():
        pltpu.sync_copy(acc, agg_hi)


BR = 1000


def _tc_body(x_ref, alo_ref, ahi_ref, ws_ref, wn_ref, bs_ref, bn_ref, o_ref):
    dn = (((1,), (1,)), ((), ()))
    h = lax.dot_general(x_ref[...], ws_ref[...], dn,
                        preferred_element_type=jnp.float32)
    wn = wn_ref[...]
    h = h + lax.dot_general(alo_ref[...], wn[:, :HALF], dn,
                            preferred_element_type=jnp.float32)
    h = h + lax.dot_general(ahi_ref[...], wn[:, HALF:], dn,
                            preferred_element_type=jnp.float32)
    o_ref[...] = jnp.maximum(h + bs_ref[...] + bn_ref[...], 0.0)


_tc_linear = pl.pallas_call(
    _tc_body,
    grid=(N_NODES // BR,),
    in_specs=[
        pl.BlockSpec((BR, DIM), lambda i: (i, 0)),
        pl.BlockSpec((BR, HALF), lambda i: (i, 0)),
        pl.BlockSpec((BR, HALF), lambda i: (i, 0)),
        pl.BlockSpec((DIM, DIM), lambda i: (0, 0)),
        pl.BlockSpec((DIM, DIM), lambda i: (0, 0)),
        pl.BlockSpec((1, DIM), lambda i: (0, 0)),
        pl.BlockSpec((1, DIM), lambda i: (0, 0)),
    ],
    out_specs=pl.BlockSpec((BR, DIM), lambda i: (i, 0)),
    out_shape=jax.ShapeDtypeStruct((N_NODES, DIM), jnp.float32),
)


def kernel(x, edge_index, W_self, b_self, W_neigh, b_neigh):
    ei = edge_index.astype(jnp.int32)
    x_lo = x[:, :HALF]
    x_hi = x[:, HALF:]
    zeros = jnp.zeros((N_NODES, HALF), jnp.float32)
    agg_lo, agg_hi = _sc_aggregate(x_lo, x_hi, ei[0], ei[1], zeros)
    return _tc_linear(x, agg_lo, agg_hi, W_self, W_neigh,
                      b_self.reshape(1, DIM), b_neigh.reshape(1, DIM))


# speedup vs baseline: 9.5092x; 1.0059x over previous
import functools

import jax
import jax.numpy as jnp
from jax import lax
from jax.experimental import pallas as pl
from jax.experimental.pallas import tpu as pltpu
from jax.experimental.pallas import tpu_sc as plsc

N_NODES = 10000
DIM = 256
HALF = 128
N_EDGES = 160000
NC = 2
NS = 16

CHUNK = 40
EDGES_PER_TILE = N_EDGES // NS
CHUNKS_PER_TILE = EDGES_PER_TILE // CHUNK
NBUF = 5

WB_ROWS = 640
WB_LAST = N_NODES - 15 * WB_ROWS

_sc_mesh = plsc.VectorSubcoreMesh(
    core_axis_name="c", subcore_axis_name="s", num_cores=NC, num_subcores=NS
)


@functools.partial(
    pl.kernel,
    out_type=(
        jax.ShapeDtypeStruct((N_NODES, HALF), jnp.float32),
        jax.ShapeDtypeStruct((N_NODES, HALF), jnp.float32),
    ),
    mesh=_sc_mesh,
    scratch_types=(
        [
            pltpu.VMEM_SHARED((N_NODES, HALF), jnp.float32),
            pltpu.VMEM((EDGES_PER_TILE,), jnp.int32),
        ]
        + [pltpu.VMEM((CHUNK, HALF), jnp.float32)] * NBUF
        + [pltpu.VMEM((CHUNK,), jnp.int32)] * NBUF
        + [pltpu.SemaphoreType.DMA] * (3 * NBUF)
    ),
)
def _sc_aggregate(x_lo, x_hi, row_hbm, col_hbm, zeros_hbm,
                  agg_lo, agg_hi, acc, col1d, *bufs_and_sems):
    rows = bufs_and_sems[:NBUF]
    rbufs = bufs_and_sems[NBUF:2 * NBUF]
    rsems = bufs_and_sems[2 * NBUF:3 * NBUF]
    gsems = bufs_and_sems[3 * NBUF:4 * NBUF]
    ssems = bufs_and_sems[4 * NBUF:]
    c = lax.axis_index("c")
    s = lax.axis_index("s")
    ebase = s * EDGES_PER_TILE

    pltpu.sync_copy(col_hbm.at[pl.ds(ebase, EDGES_PER_TILE)], col1d)

    @pl.when(s == 0)
    def _():
        pltpu.sync_copy(zeros_hbm, acc)

    plsc.subcore_barrier()

    def run(x_half):
        def issue(k, b):
            pltpu.async_copy(row_hbm.at[pl.ds(ebase + k * CHUNK, CHUNK)],
                             rbufs[b], rsems[b])
            pltpu.async_copy(x_half.at[col1d.at[pl.ds(k * CHUNK, CHUNK)]],
                             rows[b], gsems[b])

        for b in range(NBUF):
            issue(b, b)

        def outer(g, _):
            for b in range(NBUF):
                k = g * NBUF + b
                pltpu.make_async_copy(x_half.at[col1d.at[pl.ds(0, CHUNK)]],
                                      rows[b], gsems[b]).wait()
                pltpu.make_async_copy(row_hbm.at[pl.ds(0, CHUNK)],
                                      rbufs[b], rsems[b]).wait()
                pltpu.async_copy(rows[b], acc.at[rbufs[b]], ssems[b],
                                 add=True)
                pltpu.make_async_copy(rows[b], acc.at[rbufs[b]],
                                      ssems[b]).wait()

                @pl.when(k + NBUF < CHUNKS_PER_TILE)
                def _():
                    issue(k + NBUF, b)
            return 0

        lax.fori_loop(0, CHUNKS_PER_TILE // NBUF, outer, 0)

    @pl.when(c == 0)
    def _():
        run(x_lo)

    @pl.when(c == 1)
    def _():
        run(x_hi)

    plsc.subcore_barrier()

    @pl.when((s == 0) & (c == 0))
    def _():
        pltpu.sync_copy(acc, agg_lo)

    @pl.when((s == 0) & (c == 1))
    def _():
        pltpu.sync_copy(acc, agg_hi)


BR = 1000


def _tc_pre_body(x_ref, ws_ref, bs_ref, bn_ref, h_ref):
    dn = (((1,), (1,)), ((), ()))
    h = lax.dot_general(x_ref[...], ws_ref[...], dn,
                        preferred_element_type=jnp.float32)
    h_ref[...] = h + bs_ref[...] + bn_ref[...]


_tc_pre = pl.pallas_call(
    _tc_pre_body,
    grid=(N_NODES // BR,),
    in_specs=[
        pl.BlockSpec((BR, DIM), lambda i: (i, 0)),
        pl.BlockSpec((DIM, DIM), lambda i: (0, 0)),
        pl.BlockSpec((1, DIM), lambda i: (0, 0)),
        pl.BlockSpec((1, DIM), lambda i: (0, 0)),
    ],
    out_specs=pl.BlockSpec((BR, DIM), lambda i: (i, 0)),
    out_shape=jax.ShapeDtypeStruct((N_NODES, DIM), jnp.float32),
)


def _tc_post_body(h_ref, alo_ref, ahi_ref, wn_ref, o_ref):
    dn = (((1,), (1,)), ((), ()))
    wn = wn_ref[...]
    h = h_ref[...]
    h = h + lax.dot_general(alo_ref[...], wn[:, :HALF], dn,
                            preferred_element_type=jnp.float32)
    h = h + lax.dot_general(ahi_ref[...], wn[:, HALF:], dn,
                            preferred_element_type=jnp.float32)
    o_ref[...] = jnp.maximum(h, 0.0)


_tc_post = pl.pallas_call(
    _tc_post_body,
    grid=(N_NODES // BR,),
    in_specs=[
        pl.BlockSpec((BR, DIM), lambda i: (i, 0)),
        pl.BlockSpec((BR, HALF), lambda i: (i, 0)),
        pl.BlockSpec((BR, HALF), lambda i: (i, 0)),
        pl.BlockSpec((DIM, DIM), lambda i: (0, 0)),
    ],
    out_specs=pl.BlockSpec((BR, DIM), lambda i: (i, 0)),
    out_shape=jax.ShapeDtypeStruct((N_NODES, DIM), jnp.float32),
)


def kernel(x, edge_index, W_self, b_self, W_neigh, b_neigh):
    ei = edge_index.astype(jnp.int32)
    x_lo = x[:, :HALF]
    x_hi = x[:, HALF:]
    zeros = jnp.zeros((N_NODES, HALF), jnp.float32)
    agg_lo, agg_hi = _sc_aggregate(x_lo, x_hi, ei[0], ei[1], zeros)
    h = _tc_pre(x, W_self, b_self.reshape(1, DIM), b_neigh.reshape(1, DIM))
    return _tc_post(h, agg_lo, agg_hi, W_neigh)
